# baseline (speedup 1.0000x reference)
import jax
import jax.numpy as jnp
from jax.experimental import pallas as pl


def _add_body(x_ref, e_ref, o_ref):
    o_ref[...] = x_ref[...] + e_ref[...]


def kernel(inputs, embeddings):
    B, S, D = inputs.shape
    pos = embeddings[:S]
    SBLK = 1024
    n_sblk = S // SBLK
    return pl.pallas_call(
        _add_body,
        grid=(n_sblk, B),
        in_specs=[
            pl.BlockSpec((1, SBLK, D), lambda s, b: (b, s, 0)),
            pl.BlockSpec((SBLK, D), lambda s, b: (s, 0)),
        ],
        out_specs=pl.BlockSpec((1, SBLK, D), lambda s, b: (b, s, 0)),
        out_shape=jax.ShapeDtypeStruct((B, S, D), inputs.dtype),
    )(inputs, pos)


# speedup vs baseline: 1.0430x; 1.0430x over previous
import jax
import jax.numpy as jnp
from jax.experimental import pallas as pl


def _add_body(x_ref, e_ref, o_ref):
    o_ref[...] = x_ref[...] + e_ref[...]


def kernel(inputs, embeddings):
    B, S, D = inputs.shape
    pos = embeddings[:S]
    SBLK = 2048
    n_sblk = S // SBLK
    return pl.pallas_call(
        _add_body,
        grid=(n_sblk, B),
        in_specs=[
            pl.BlockSpec((1, SBLK, D), lambda s, b: (b, s, 0)),
            pl.BlockSpec((SBLK, D), lambda s, b: (s, 0)),
        ],
        out_specs=pl.BlockSpec((1, SBLK, D), lambda s, b: (b, s, 0)),
        out_shape=jax.ShapeDtypeStruct((B, S, D), inputs.dtype),
    )(inputs, pos)
